# Initial kernel scaffold; baseline (speedup 1.0000x reference)
import jax
import jax.numpy as jnp
from jax.experimental import pallas as pl


def kernel(ctx, cnt_table, val_table, W, b):
    raise NotImplementedError("write your pallas kernel here")


# speedup vs baseline: 1.0677x; 1.0677x over previous
import jax
import jax.numpy as jnp
from jax import lax
from jax.experimental import pallas as pl
from jax.experimental.pallas import tpu as pltpu
from jax.experimental.pallas import tpu_sc as plsc

_K = 26
_NEMBED = 32
_NHID = 128
_B = 16384
_D = _K * _NEMBED

_INFO = plsc.get_sparse_core_info()
_NC = _INFO.num_cores
_NS = _INFO.num_subcores
_NW = _NC * _NS
_CB = 64
_NCH = _B // (_NW * _CB)
_GR = _CB * _K


def _sc_body(cids_hbm, vids_hbm, cnt_hbm, val_hbm, h_hbm,
             idx_c, idx_v, rows_c, rows_v, sem):
    wid = lax.axis_index("s") * _NC + lax.axis_index("c")

    def per_chunk(c, carry):
        i0 = pl.multiple_of((wid * _NCH + c) * _GR, _GR)
        pltpu.sync_copy(cids_hbm.at[pl.ds(i0, _GR)], idx_c)
        pltpu.sync_copy(vids_hbm.at[pl.ds(i0, _GR)], idx_v)
        cpy_c = pltpu.async_copy(cnt_hbm.at[idx_c], rows_c, sem)
        cpy_v = pltpu.async_copy(val_hbm.at[idx_v], rows_v, sem)
        cpy_c.wait()
        cpy_v.wait()

        def mul_row(r, carry2):
            for j in (0, 16):
                rows_c[r, pl.ds(j, 16)] = (
                    rows_c[r, pl.ds(j, 16)] * rows_v[r, pl.ds(j, 16)]
                )
            return carry2

        lax.fori_loop(0, _GR, mul_row, 0, unroll=4)
        pltpu.sync_copy(rows_c, h_hbm.at[pl.ds(i0, _GR), :])
        return carry

    lax.fori_loop(0, _NCH, per_chunk, 0)


def _sc_gather_mul(cids, vids, cnt_table, val_table):
    mesh = plsc.VectorSubcoreMesh(core_axis_name="c", subcore_axis_name="s")
    f = pl.kernel(
        _sc_body,
        out_type=jax.ShapeDtypeStruct((_B * _K, _NEMBED), jnp.float32),
        mesh=mesh,
        compiler_params=pltpu.CompilerParams(use_tc_tiling_on_sc=False),
        scratch_types=[
            pltpu.VMEM((_GR,), jnp.int32),
            pltpu.VMEM((_GR,), jnp.int32),
            pltpu.VMEM((_GR, _NEMBED), jnp.float32),
            pltpu.VMEM((_GR, _NEMBED), jnp.float32),
            pltpu.SemaphoreType.DMA,
        ],
    )
    return f(cids, vids, cnt_table, val_table)


def _tc_body(h_ref, w_ref, b_ref, o_ref):
    o_ref[...] = (
        jnp.dot(jnp.tanh(h_ref[...]), w_ref[...], preferred_element_type=jnp.float32)
        + b_ref[...]
    )


def _tc_mlp(h, W, b):
    mb = 2048
    return pl.pallas_call(
        _tc_body,
        grid=(_B // mb,),
        in_specs=[
            pl.BlockSpec((mb, _D), lambda i: (i, 0)),
            pl.BlockSpec((_D, _NHID), lambda i: (0, 0)),
            pl.BlockSpec((1, _NHID), lambda i: (0, 0)),
        ],
        out_specs=pl.BlockSpec((mb, _NHID), lambda i: (i, 0)),
        out_shape=jax.ShapeDtypeStruct((_B, _NHID), jnp.float32),
    )(h, W, b.reshape(1, _NHID))


@jax.jit
def kernel(ctx, cnt_table, val_table, W, b):
    cids = ctx[0::2].T.reshape(-1)
    vids = ctx[1::2].T.reshape(-1)
    h = _sc_gather_mul(cids, vids, cnt_table, val_table).reshape(_B, _D)
    out = _tc_mlp(h, W, b)
    return out[None, :, :]
